# baseline (speedup 1.0000x reference)
import functools

import jax
import jax.numpy as jnp
from jax import lax
from jax.experimental import pallas as pl
from jax.experimental.pallas import tpu as pltpu
from jax.experimental.pallas import tpu_sc as plsc

N_VOCAB = 49408
N_EMBED = 768
N_TOKENS = 77
BATCH = 128

_NC = 2
_NS = 16
_NW = _NC * _NS
_ROWS_PER_W = BATCH // _NW
_LANES = 16
_PAD_TOKENS = 80
_VECS_PER_ROW = N_EMBED // _LANES


def _make_sc_lookup():
    mesh = plsc.VectorSubcoreMesh(core_axis_name="c", subcore_axis_name="s")

    @functools.partial(
        pl.kernel,
        mesh=mesh,
        compiler_params=pltpu.CompilerParams(use_tc_tiling_on_sc=False),
        out_type=jax.ShapeDtypeStruct((BATCH, N_TOKENS, N_EMBED), jnp.float32),
        scratch_types=[
            pltpu.VMEM((_ROWS_PER_W, _PAD_TOKENS), jnp.int32),
            pltpu.VMEM((N_TOKENS, N_EMBED), jnp.float32),
            pltpu.VMEM((_PAD_TOKENS, N_EMBED), jnp.float32),
            pltpu.SemaphoreType.DMA,
        ],
    )
    def lookup(tok_hbm, table_hbm, pos_hbm, out_hbm, idx_v, pos_v, buf_v, sem):
        wid = lax.axis_index("s") * _NC + lax.axis_index("c")
        pltpu.sync_copy(tok_hbm.at[wid], idx_v)
        pltpu.sync_copy(pos_hbm, pos_v)
        for c in range(_ROWS_PER_W):
            pltpu.async_copy(table_hbm.at[idx_v.at[c]], buf_v, sem).wait()
            pltpu.sync_copy(buf_v.at[pl.ds(0, N_TOKENS)],
                            out_hbm.at[_ROWS_PER_W * wid + c])

    return lookup


_sc_lookup = _make_sc_lookup()


def kernel(tokens, token_embedding, position_embedding):
    tok32 = tokens.astype(jnp.int32).reshape(_NW, _ROWS_PER_W, N_TOKENS)
    tok32 = jnp.pad(tok32, ((0, 0), (0, 0), (0, _PAD_TOKENS - N_TOKENS)))
    return _sc_lookup(tok32, token_embedding, position_embedding)


# speedup vs baseline: 1.0492x; 1.0492x over previous
import functools

import jax
import jax.numpy as jnp
from jax import lax
from jax.experimental import pallas as pl
from jax.experimental.pallas import tpu as pltpu
from jax.experimental.pallas import tpu_sc as plsc

N_VOCAB = 49408
N_EMBED = 768
N_TOKENS = 77
BATCH = 128

_NC = 2
_NS = 16
_NW = _NC * _NS
_ROWS_PER_W = BATCH // _NW
_LANES = 16
_PAD_TOKENS = 80
_VECS_PER_ROW = N_EMBED // _LANES


def _make_sc_lookup():
    mesh = plsc.VectorSubcoreMesh(core_axis_name="c", subcore_axis_name="s")

    @functools.partial(
        pl.kernel,
        mesh=mesh,
        compiler_params=pltpu.CompilerParams(use_tc_tiling_on_sc=False),
        out_type=jax.ShapeDtypeStruct((BATCH, N_TOKENS, N_EMBED), jnp.float32),
        scratch_types=[
            pltpu.VMEM((_ROWS_PER_W, _PAD_TOKENS), jnp.int32),
            pltpu.VMEM((_PAD_TOKENS, N_EMBED), jnp.float32),
            pltpu.VMEM((_PAD_TOKENS, N_EMBED), jnp.float32),
            pltpu.SemaphoreType.DMA,
            pltpu.SemaphoreType.DMA,
        ],
    )
    def lookup(tok_hbm, table_hbm, pos_hbm, out_hbm, idx_v, buf_a, buf_b,
               gsem, wsem):
        wid = lax.axis_index("s") * _NC + lax.axis_index("c")
        pltpu.sync_copy(tok_hbm.at[wid], idx_v)
        bufs = (buf_a, buf_b)
        gathers = [None, None]
        writes = [None, None]
        for c in range(_ROWS_PER_W):
            if writes[c % 2] is not None:
                writes[c % 2].wait()
            gathers[c % 2] = pltpu.async_copy(
                table_hbm.at[idx_v.at[c]], bufs[c % 2], gsem)
            if c >= 1:
                p = (c - 1) % 2
                gathers[p].wait()
                writes[p] = pltpu.async_copy(
                    bufs[p].at[pl.ds(0, N_TOKENS)],
                    out_hbm.at[_ROWS_PER_W * wid + c - 1], wsem)
        last = (_ROWS_PER_W - 1) % 2
        gathers[last].wait()
        writes[last] = pltpu.async_copy(
            bufs[last].at[pl.ds(0, N_TOKENS)],
            out_hbm.at[_ROWS_PER_W * wid + _ROWS_PER_W - 1], wsem)
        for w in writes:
            w.wait()

    return lookup


_sc_lookup = _make_sc_lookup()


def kernel(tokens, token_embedding, position_embedding):
    tok32 = tokens.astype(jnp.int32).reshape(_NW, _ROWS_PER_W, N_TOKENS)
    tok32 = jnp.pad(tok32, ((0, 0), (0, 0), (0, _PAD_TOKENS - N_TOKENS)))
    return _sc_lookup(tok32, token_embedding, position_embedding)


# speedup vs baseline: 1.9511x; 1.8596x over previous
import functools

import jax
import jax.numpy as jnp
from jax import lax
from jax.experimental import pallas as pl
from jax.experimental.pallas import tpu as pltpu
from jax.experimental.pallas import tpu_sc as plsc

N_VOCAB = 49408
N_EMBED = 768
N_TOKENS = 77
BATCH = 128

_NC = 2
_NS = 16
_NW = _NC * _NS
_ROWS_PER_W = BATCH // _NW
_LANES = 16
_PAD_TOKENS = 80
_CHUNK = 16
_NCHUNK = _PAD_TOKENS // _CHUNK
_PAIRS = N_EMBED // 32


def _make_sc_lookup():
    mesh = plsc.VectorSubcoreMesh(core_axis_name="c", subcore_axis_name="s")

    @functools.partial(
        pl.kernel,
        mesh=mesh,
        out_type=jax.ShapeDtypeStruct((BATCH, N_TOKENS, N_EMBED), jnp.float32),
        scratch_types=[
            pltpu.VMEM((_ROWS_PER_W, _PAD_TOKENS), jnp.int32),
            pltpu.VMEM((N_TOKENS * N_EMBED // 2,), jnp.int32),
            pltpu.VMEM((_CHUNK, N_EMBED), jnp.float32),
            pltpu.VMEM((_CHUNK, N_EMBED), jnp.float32),
            pltpu.VMEM((N_TOKENS, N_EMBED), jnp.float32),
            pltpu.SemaphoreType.DMA,
            pltpu.SemaphoreType.DMA,
            pltpu.SemaphoreType.DMA,
        ],
    )
    def lookup(tok_hbm, table_hbm, pos_hbm, out_hbm,
               idx_v, pos_v, buf_a, buf_b, wbuf,
               gsem, wsem, psem):
        wid = lax.axis_index("s") * _NC + lax.axis_index("c")
        pos_dma = pltpu.async_copy(pos_hbm, pos_v, psem)
        pltpu.sync_copy(tok_hbm.at[wid], idx_v)
        gbufs = (buf_a, buf_b)

        def gather(b, p, buf):
            return pltpu.async_copy(
                table_hbm.at[idx_v.at[b, pl.ds(_CHUNK * p, _CHUNK)]],
                buf, gsem)

        def add_chunk(p, nrows):
            gbuf = gbufs[p % 2]

            def row_body(r, carry):
                t = _CHUNK * p + r
                for j in range(_PAIRS):
                    off = pl.multiple_of(t * (N_EMBED // 2) + 16 * j, 16)
                    w = pos_v[pl.ds(off, _LANES)]
                    lo = lax.bitcast_convert_type(w << 16, jnp.float32)
                    hi = lax.bitcast_convert_type(w & jnp.int32(-65536),
                                                  jnp.float32)
                    sa = pl.ds(32 * j, _LANES)
                    sb = pl.ds(32 * j + _LANES, _LANES)
                    wbuf[t, sa] = gbuf[r, sa] + lo
                    wbuf[t, sb] = gbuf[r, sb] + hi
                return carry

            lax.fori_loop(0, nrows, row_body, 0)

        pos_dma.wait()
        write = None
        for b in range(_ROWS_PER_W):
            gather(b, 0, gbufs[0]).wait()
            g_next = gather(b, 1, gbufs[1])
            if write is not None:
                write.wait()
            for p in range(_NCHUNK):
                if p > 0:
                    g_next.wait()
                    if p + 1 < _NCHUNK:
                        g_next = gather(b, p + 1, gbufs[(p + 1) % 2])
                    elif b + 1 < _ROWS_PER_W:
                        g_next = None
                add_chunk(p, _CHUNK if p + 1 < _NCHUNK
                          else N_TOKENS - _CHUNK * (_NCHUNK - 1))
            write = pltpu.async_copy(
                wbuf, out_hbm.at[_ROWS_PER_W * wid + b], wsem)
        write.wait()

    return lookup


_sc_lookup = _make_sc_lookup()


def kernel(tokens, token_embedding, position_embedding):
    tok32 = tokens.astype(jnp.int32).reshape(_NW, _ROWS_PER_W, N_TOKENS)
    tok32 = jnp.pad(tok32, ((0, 0), (0, 0), (0, _PAD_TOKENS - N_TOKENS)))
    pos_bf = (position_embedding.reshape(N_TOKENS, _PAIRS, 2, _LANES)
              .swapaxes(-2, -1)
              .reshape(N_TOKENS * N_EMBED // 2, 2)
              .astype(jnp.bfloat16))
    pos_pk = jax.lax.bitcast_convert_type(pos_bf, jnp.int32)
    return _sc_lookup(tok32, token_embedding, pos_pk)


# speedup vs baseline: 2.3740x; 1.2168x over previous
import functools

import jax
import jax.numpy as jnp
from jax import lax
from jax.experimental import pallas as pl
from jax.experimental.pallas import tpu as pltpu
from jax.experimental.pallas import tpu_sc as plsc

N_VOCAB = 49408
N_EMBED = 768
N_TOKENS = 77
BATCH = 128

_NC = 2
_NS = 16
_NW = _NC * _NS
_ROWS_PER_W = BATCH // _NW
_LANES = 16
_PAD_TOKENS = 80
_CHUNK = 16
_NCHUNK = _PAD_TOKENS // _CHUNK
_PAIRS = N_EMBED // 32


def _make_sc_lookup():
    mesh = plsc.VectorSubcoreMesh(core_axis_name="c", subcore_axis_name="s")

    @functools.partial(
        pl.kernel,
        mesh=mesh,
        out_type=jax.ShapeDtypeStruct((BATCH, N_TOKENS, N_EMBED), jnp.float32),
        scratch_types=[
            pltpu.VMEM((_ROWS_PER_W, _PAD_TOKENS), jnp.int32),
            pltpu.VMEM((N_TOKENS * N_EMBED // 2,), jnp.int32),
            pltpu.VMEM((_CHUNK, N_EMBED), jnp.float32),
            pltpu.VMEM((_CHUNK, N_EMBED), jnp.float32),
            pltpu.VMEM((N_TOKENS, N_EMBED), jnp.float32),
            pltpu.SemaphoreType.DMA,
            pltpu.SemaphoreType.DMA,
            pltpu.SemaphoreType.DMA,
        ],
    )
    def lookup(tok_hbm, table_hbm, pos_hbm, out_hbm,
               idx_v, pos_v, buf_a, buf_b, wbuf,
               gsem, wsem, psem):
        wid = lax.axis_index("s") * _NC + lax.axis_index("c")
        pos_dma = pltpu.async_copy(pos_hbm, pos_v, psem)
        pltpu.sync_copy(tok_hbm.at[wid], idx_v)
        gbufs = (buf_a, buf_b)

        def gather(b, p, buf):
            return pltpu.async_copy(
                table_hbm.at[idx_v.at[b, pl.ds(_CHUNK * p, _CHUNK)]],
                buf, gsem)

        def add_chunk(p, nrows):
            gbuf = gbufs[p % 2]

            @plsc.parallel_loop(0, nrows)
            def row_body(r):
                t = _CHUNK * p + r
                for j in range(_PAIRS):
                    off = pl.multiple_of(t * (N_EMBED // 2) + 16 * j, 16)
                    w = pos_v[pl.ds(off, _LANES)]
                    lo = lax.bitcast_convert_type(w << 16, jnp.float32)
                    hi = lax.bitcast_convert_type(w & jnp.int32(-65536),
                                                  jnp.float32)
                    sa = pl.ds(32 * j, _LANES)
                    sb = pl.ds(32 * j + _LANES, _LANES)
                    wbuf[t, sa] = gbuf[r, sa] + lo
                    wbuf[t, sb] = gbuf[r, sb] + hi

        pos_dma.wait()

        def wait_write():
            pltpu.make_async_copy(wbuf, out_hbm.at[0], wsem).wait()

        def batch_body(b, carry):
            g_next = [gather(b, 0, gbufs[0])]

            def issue(p):
                g_next[0] = gather(b, p, gbufs[p % 2])

            g0 = g_next[0]
            issue(1)
            g1 = g_next[0]
            g0.wait()

            @pl.when(b > 0)
            def _():
                wait_write()

            handles = [g0, g1]
            for p in range(_NCHUNK):
                if p > 0:
                    handles[p].wait()
                    if p + 1 < _NCHUNK:
                        issue(p + 1)
                        handles.append(g_next[0])
                add_chunk(p, _CHUNK if p + 1 < _NCHUNK
                          else N_TOKENS - _CHUNK * (_NCHUNK - 1))
            pltpu.async_copy(wbuf, out_hbm.at[_ROWS_PER_W * wid + b], wsem)
            return carry

        lax.fori_loop(0, _ROWS_PER_W, batch_body, 0)
        wait_write()

    return lookup


_sc_lookup = _make_sc_lookup()


def kernel(tokens, token_embedding, position_embedding):
    tok32 = tokens.astype(jnp.int32).reshape(_NW, _ROWS_PER_W, N_TOKENS)
    tok32 = jnp.pad(tok32, ((0, 0), (0, 0), (0, _PAD_TOKENS - N_TOKENS)))
    pos_bf = (position_embedding.reshape(N_TOKENS, _PAIRS, 2, _LANES)
              .swapaxes(-2, -1)
              .reshape(N_TOKENS * N_EMBED // 2, 2)
              .astype(jnp.bfloat16))
    pos_pk = jax.lax.bitcast_convert_type(pos_bf, jnp.int32)
    return _sc_lookup(tok32, token_embedding, pos_pk)


# speedup vs baseline: 2.4726x; 1.0415x over previous
import functools

import jax
import jax.numpy as jnp
from jax import lax
from jax.experimental import pallas as pl
from jax.experimental.pallas import tpu as pltpu
from jax.experimental.pallas import tpu_sc as plsc

N_VOCAB = 49408
N_EMBED = 768
N_TOKENS = 77
BATCH = 128

_NC = 2
_NS = 16
_NW = _NC * _NS
_ROWS_PER_W = BATCH // _NW
_LANES = 16
_PAD_TOKENS = 80
_CHUNK = 16
_NCHUNK = _PAD_TOKENS // _CHUNK
_PAIRS = N_EMBED // 32


def _make_sc_lookup():
    mesh = plsc.VectorSubcoreMesh(core_axis_name="c", subcore_axis_name="s")

    @functools.partial(
        pl.kernel,
        mesh=mesh,
        out_type=jax.ShapeDtypeStruct((BATCH, N_TOKENS, N_EMBED), jnp.float32),
        scratch_types=[
            pltpu.VMEM((_ROWS_PER_W, _PAD_TOKENS), jnp.int32),
            pltpu.VMEM((N_TOKENS * N_EMBED // 2,), jnp.int32),
            pltpu.VMEM((_CHUNK, N_EMBED), jnp.float32),
            pltpu.VMEM((_CHUNK, N_EMBED), jnp.float32),
            pltpu.VMEM((_CHUNK, N_EMBED), jnp.float32),
            pltpu.VMEM((N_TOKENS, N_EMBED), jnp.float32),
            pltpu.SemaphoreType.DMA,
            pltpu.SemaphoreType.DMA,
            pltpu.SemaphoreType.DMA,
        ],
    )
    def lookup(tok_hbm, table_hbm, pos_hbm, out_hbm,
               idx_v, pos_v, buf_a, buf_b, buf_c, wbuf,
               gsem, wsem, psem):
        wid = lax.axis_index("s") * _NC + lax.axis_index("c")
        pos_dma = pltpu.async_copy(pos_hbm, pos_v, psem)
        pltpu.sync_copy(tok_hbm.at[wid], idx_v)
        gbufs = (buf_a, buf_b, buf_c)

        def gather(b, p, buf):
            return pltpu.async_copy(
                table_hbm.at[idx_v.at[b, pl.ds(_CHUNK * p, _CHUNK)]],
                buf, gsem)

        def add_chunk(p, nrows):
            gbuf = gbufs[p % 3]

            @plsc.parallel_loop(0, nrows)
            def row_body(r):
                t = _CHUNK * p + r
                for j in range(_PAIRS):
                    off = pl.multiple_of(t * (N_EMBED // 2) + 16 * j, 16)
                    w = pos_v[pl.ds(off, _LANES)]
                    lo = lax.bitcast_convert_type(w << 16, jnp.float32)
                    hi = lax.bitcast_convert_type(w & jnp.int32(-65536),
                                                  jnp.float32)
                    sa = pl.ds(32 * j, _LANES)
                    sb = pl.ds(32 * j + _LANES, _LANES)
                    wbuf[t, sa] = gbuf[r, sa] + lo
                    wbuf[t, sb] = gbuf[r, sb] + hi

        pos_dma.wait()

        def wait_write():
            pltpu.make_async_copy(wbuf, out_hbm.at[0], wsem).wait()

        def batch_body(b, carry):
            def gat(p):
                return gather(b, p, gbufs[p % 3])

            handles = [gat(0), gat(1), gat(2)]

            @pl.when(b > 0)
            def _():
                wait_write()

            for p in range(_NCHUNK):
                handles[p].wait()
                if p + 3 < _NCHUNK:
                    handles.append(gat(p + 3))
                add_chunk(p, _CHUNK if p + 1 < _NCHUNK
                          else N_TOKENS - _CHUNK * (_NCHUNK - 1))
            pltpu.async_copy(wbuf, out_hbm.at[_ROWS_PER_W * wid + b], wsem)
            return carry

        lax.fori_loop(0, _ROWS_PER_W, batch_body, 0)
        wait_write()

    return lookup


_sc_lookup = _make_sc_lookup()


def kernel(tokens, token_embedding, position_embedding):
    tok32 = tokens.astype(jnp.int32).reshape(_NW, _ROWS_PER_W, N_TOKENS)
    tok32 = jnp.pad(tok32, ((0, 0), (0, 0), (0, _PAD_TOKENS - N_TOKENS)))
    pos_bf = (position_embedding.reshape(N_TOKENS, _PAIRS, 2, _LANES)
              .swapaxes(-2, -1)
              .reshape(N_TOKENS * N_EMBED // 2, 2)
              .astype(jnp.bfloat16))
    pos_pk = jax.lax.bitcast_convert_type(pos_bf, jnp.int32)
    return _sc_lookup(tok32, token_embedding, pos_pk)
